# baseline (speedup 1.0000x reference)
import jax
import jax.numpy as jnp
from jax import lax
from jax.experimental import pallas as pl
from jax.experimental.pallas import tpu as pltpu
from jax.experimental.pallas import tpu_sc as plsc

N_NODE = 100000
D = 32
E = 1600000
HALF = 50000
ACC_ROWS = 50176
DUMMY = HALF
EPT = E // 16
BLK = 800
NBLK = EPT // BLK
CH = 80
NCH = BLK // CH


def _sc_body(src_e, dst_e, user_x, item_x, zrow, orow,
             sum_i, cnt_i, sum_u, cnt_u,
             gbuf, dbuf, sidx, rows, ones, acc):
    c = lax.axis_index("c")
    s = lax.axis_index("s")
    lo = c * HALF

    pltpu.sync_copy(orow, ones)

    def zero_acc():
        pltpu.sync_copy(zrow.at[pl.ds(0, 3136)],
                        acc.at[pl.ds(s * 3136, 3136)])

    def scan_pass(s_arr, g_arr, table):
        @pl.loop(0, NBLK)
        def blk_body(blk):
            base = s * EPT + blk * BLK
            if table is not None:
                pltpu.sync_copy(g_arr.at[pl.ds(base, BLK)], gbuf)
            pltpu.sync_copy(s_arr.at[pl.ds(base, BLK)], dbuf)

            @pl.loop(0, NCH)
            def scan_ch(t):
                for q in range(CH // 16):
                    dv = dbuf[pl.ds(t * CH + q * 16, 16)]
                    ld = dv - lo
                    m = (ld >= 0) & (ld < HALF)
                    sidx[t, pl.ds(q * 16, 16)] = jnp.where(m, ld, DUMMY)

            @pl.loop(0, NCH)
            def main_ch(t):
                if table is not None:
                    pltpu.sync_copy(table.at[gbuf.at[pl.ds(t * CH, CH)]],
                                    rows)
                    pltpu.sync_copy(rows, acc.at[sidx.at[t]], add=True)
                else:
                    pltpu.sync_copy(ones, acc.at[sidx.at[t]], add=True)

    def write_out(sum_o):
        @pl.loop(0, 40)
        def wrs(k):
            j = k * 16 + s

            @pl.when(j < HALF // 80)
            def _():
                pltpu.sync_copy(acc.at[pl.ds(j * 80, 80)],
                                sum_o.at[pl.ds(c * HALF + j * 80, 80)])

    def phase(s_arr, g_arr, table, sum_o):
        zero_acc()
        plsc.subcore_barrier()
        scan_pass(s_arr, g_arr, table)
        plsc.subcore_barrier()
        write_out(sum_o)
        plsc.subcore_barrier()

    phase(dst_e, src_e, user_x, sum_i)
    phase(dst_e, None, None, cnt_i)
    phase(src_e, dst_e, item_x, sum_u)
    phase(src_e, None, None, cnt_u)


_sc_call = pl.kernel(
    _sc_body,
    out_type=(
        jax.ShapeDtypeStruct((N_NODE, D), jnp.float32),
        jax.ShapeDtypeStruct((N_NODE, D), jnp.float32),
        jax.ShapeDtypeStruct((N_NODE, D), jnp.float32),
        jax.ShapeDtypeStruct((N_NODE, D), jnp.float32),
    ),
    mesh=plsc.VectorSubcoreMesh(core_axis_name="c", subcore_axis_name="s"),
    compiler_params=pltpu.CompilerParams(use_tc_tiling_on_sc=False),
    scratch_types=(
        pltpu.VMEM((BLK,), jnp.int32),
        pltpu.VMEM((BLK,), jnp.int32),
        pltpu.VMEM((NCH, CH), jnp.int32),
        pltpu.VMEM((CH, D), jnp.float32),
        pltpu.VMEM((CH, D), jnp.float32),
        pltpu.VMEM_SHARED((ACC_ROWS, D), jnp.float32),
    ),
)

BR = 1000


def _dense_body(hu, su, cu, hi, si, ci, wsT, wnT, bb, ou, oi):
    ws = wsT[...]
    wn = wnT[...]
    bv = bb[...]
    agg_u = su[...] * (1.0 / jnp.maximum(cu[...][:, 0:1], 1.0))
    ou[...] = (jnp.dot(hu[...], ws, preferred_element_type=jnp.float32)
               + jnp.dot(agg_u, wn, preferred_element_type=jnp.float32) + bv)
    agg_i = si[...] * (1.0 / jnp.maximum(ci[...][:, 0:1], 1.0))
    oi[...] = (jnp.dot(hi[...], ws, preferred_element_type=jnp.float32)
               + jnp.dot(agg_i, wn, preferred_element_type=jnp.float32) + bv)


def _spec(cols):
    return pl.BlockSpec((BR, cols), lambda i: (i, 0))


def _full_spec(shape):
    return pl.BlockSpec(shape, lambda i: tuple(0 for _ in shape))


_dense_call = pl.pallas_call(
    _dense_body,
    grid=(N_NODE // BR,),
    in_specs=[
        _spec(D), _spec(D), _spec(D), _spec(D), _spec(D), _spec(D),
        _full_spec((D, D)), _full_spec((D, D)), _full_spec((1, D)),
    ],
    out_specs=[_spec(D), _spec(D)],
    out_shape=(
        jax.ShapeDtypeStruct((N_NODE, D), jnp.float32),
        jax.ShapeDtypeStruct((N_NODE, D), jnp.float32),
    ),
)


def kernel(edge_index, user_table, item_table, W_self, W_neigh, b):
    edge_index = edge_index.astype(jnp.int32)
    zrow = jnp.zeros((3144, D), jnp.float32)
    orow = jnp.ones((CH, D), jnp.float32)
    sum_i, cnt_i, sum_u, cnt_u = _sc_call(
        edge_index[0], edge_index[1], user_table, item_table, zrow, orow)
    out_user, out_item = _dense_call(
        user_table, sum_u, cnt_u, item_table, sum_i, cnt_i,
        W_self.T, W_neigh.T, b.reshape(1, D))
    return (out_user, out_item)


# speedup vs baseline: 1.3149x; 1.3149x over previous
import jax
import jax.numpy as jnp
from jax import lax
from jax.experimental import pallas as pl
from jax.experimental.pallas import tpu as pltpu
from jax.experimental.pallas import tpu_sc as plsc

N_NODE = 100000
D = 32
E = 1600000
HALF = 50000
ACC_ROWS = 50176
DUMMY = HALF
EPT = E // 16
BLK = 800
NBLK = EPT // BLK
CH = 80
NCH = BLK // CH


def _sc_body(src_e, dst_e, user_x, item_x, zrow, orow,
             sum_i, cnt_i, sum_u, cnt_u,
             gbuf, dbuf, sidx, rows, ones, acc):
    c = lax.axis_index("c")
    s = lax.axis_index("s")
    lo = c * HALF

    pltpu.sync_copy(orow, ones)

    def zero_acc():
        pltpu.sync_copy(zrow.at[pl.ds(0, 3136)],
                        acc.at[pl.ds(s * 3136, 3136)])

    def scan_pass(s_arr, g_arr, table):
        @pl.loop(0, NBLK)
        def blk_body(blk):
            base = s * EPT + blk * BLK
            if table is not None:
                pltpu.sync_copy(g_arr.at[pl.ds(base, BLK)], gbuf)
            pltpu.sync_copy(s_arr.at[pl.ds(base, BLK)], dbuf)

            @pl.loop(0, NCH)
            def scan_ch(t):
                iot = jax.lax.iota(jnp.int32, 16)
                for q in range(CH // 16):
                    dv = dbuf[pl.ds(t * CH + q * 16, 16)]
                    ld = dv - lo
                    m = (ld >= 0) & (ld < HALF)
                    dum = DUMMY + ((t + q) % 10) * 16 + iot
                    sidx[t, pl.ds(q * 16, 16)] = jnp.where(m, ld, dum)

            @pl.loop(0, NCH)
            def main_ch(t):
                if table is not None:
                    pltpu.sync_copy(table.at[gbuf.at[pl.ds(t * CH, CH)]],
                                    rows)
                    pltpu.sync_copy(rows, acc.at[sidx.at[t]], add=True)
                else:
                    pltpu.sync_copy(ones, acc.at[sidx.at[t]], add=True)

    def write_out(sum_o):
        @pl.loop(0, 40)
        def wrs(k):
            j = k * 16 + s

            @pl.when(j < HALF // 80)
            def _():
                pltpu.sync_copy(acc.at[pl.ds(j * 80, 80)],
                                sum_o.at[pl.ds(c * HALF + j * 80, 80)])

    def phase(s_arr, g_arr, table, sum_o):
        zero_acc()
        plsc.subcore_barrier()
        scan_pass(s_arr, g_arr, table)
        plsc.subcore_barrier()
        write_out(sum_o)
        plsc.subcore_barrier()

    phase(dst_e, src_e, user_x, sum_i)
    phase(dst_e, None, None, cnt_i)
    phase(src_e, dst_e, item_x, sum_u)
    phase(src_e, None, None, cnt_u)


_sc_call = pl.kernel(
    _sc_body,
    out_type=(
        jax.ShapeDtypeStruct((N_NODE, D), jnp.float32),
        jax.ShapeDtypeStruct((N_NODE, D), jnp.float32),
        jax.ShapeDtypeStruct((N_NODE, D), jnp.float32),
        jax.ShapeDtypeStruct((N_NODE, D), jnp.float32),
    ),
    mesh=plsc.VectorSubcoreMesh(core_axis_name="c", subcore_axis_name="s"),
    compiler_params=pltpu.CompilerParams(use_tc_tiling_on_sc=False),
    scratch_types=(
        pltpu.VMEM((BLK,), jnp.int32),
        pltpu.VMEM((BLK,), jnp.int32),
        pltpu.VMEM((NCH, CH), jnp.int32),
        pltpu.VMEM((CH, D), jnp.float32),
        pltpu.VMEM((CH, D), jnp.float32),
        pltpu.VMEM_SHARED((ACC_ROWS, D), jnp.float32),
    ),
)

BR = 1000


def _dense_body(hu, su, cu, hi, si, ci, wsT, wnT, bb, ou, oi):
    ws = wsT[...]
    wn = wnT[...]
    bv = bb[...]
    agg_u = su[...] * (1.0 / jnp.maximum(cu[...][:, 0:1], 1.0))
    ou[...] = (jnp.dot(hu[...], ws, preferred_element_type=jnp.float32)
               + jnp.dot(agg_u, wn, preferred_element_type=jnp.float32) + bv)
    agg_i = si[...] * (1.0 / jnp.maximum(ci[...][:, 0:1], 1.0))
    oi[...] = (jnp.dot(hi[...], ws, preferred_element_type=jnp.float32)
               + jnp.dot(agg_i, wn, preferred_element_type=jnp.float32) + bv)


def _spec(cols):
    return pl.BlockSpec((BR, cols), lambda i: (i, 0))


def _full_spec(shape):
    return pl.BlockSpec(shape, lambda i: tuple(0 for _ in shape))


_dense_call = pl.pallas_call(
    _dense_body,
    grid=(N_NODE // BR,),
    in_specs=[
        _spec(D), _spec(D), _spec(D), _spec(D), _spec(D), _spec(D),
        _full_spec((D, D)), _full_spec((D, D)), _full_spec((1, D)),
    ],
    out_specs=[_spec(D), _spec(D)],
    out_shape=(
        jax.ShapeDtypeStruct((N_NODE, D), jnp.float32),
        jax.ShapeDtypeStruct((N_NODE, D), jnp.float32),
    ),
)


def kernel(edge_index, user_table, item_table, W_self, W_neigh, b):
    edge_index = edge_index.astype(jnp.int32)
    zrow = jnp.zeros((3144, D), jnp.float32)
    orow = jnp.ones((CH, D), jnp.float32)
    sum_i, cnt_i, sum_u, cnt_u = _sc_call(
        edge_index[0], edge_index[1], user_table, item_table, zrow, orow)
    out_user, out_item = _dense_call(
        user_table, sum_u, cnt_u, item_table, sum_i, cnt_i,
        W_self.T, W_neigh.T, b.reshape(1, D))
    return (out_user, out_item)


# speedup vs baseline: 1.4542x; 1.1059x over previous
import jax
import jax.numpy as jnp
from jax import lax
from jax.experimental import pallas as pl
from jax.experimental.pallas import tpu as pltpu
from jax.experimental.pallas import tpu_sc as plsc

N_NODE = 100000
D = 32
DE = 40
E = 1600000
HALF = 50000
ACC_ROWS = 50176
DUMMY = HALF
EPT = E // 16
BLK = 800
NBLK = EPT // BLK
CH = 80
NCH = BLK // CH


def _sc_body(src_e, dst_e, user_x, item_x, zrow,
             sum_i, sum_u,
             gbuf, dbuf, sidx, rows, acc):
    c = lax.axis_index("c")
    s = lax.axis_index("s")
    lo = c * HALF

    def zero_acc():
        pltpu.sync_copy(zrow.at[pl.ds(0, 3136)],
                        acc.at[pl.ds(s * 3136, 3136)])

    def scan_pass(s_arr, g_arr, table):
        @pl.loop(0, NBLK)
        def blk_body(blk):
            base = s * EPT + blk * BLK
            pltpu.sync_copy(g_arr.at[pl.ds(base, BLK)], gbuf)
            pltpu.sync_copy(s_arr.at[pl.ds(base, BLK)], dbuf)

            @pl.loop(0, NCH)
            def scan_ch(t):
                iot = jax.lax.iota(jnp.int32, 16)
                for q in range(CH // 16):
                    dv = dbuf[pl.ds(t * CH + q * 16, 16)]
                    ld = dv - lo
                    m = (ld >= 0) & (ld < HALF)
                    dum = DUMMY + ((t + q) % 10) * 16 + iot
                    sidx[t, pl.ds(q * 16, 16)] = jnp.where(m, ld, dum)

            @pl.loop(0, NCH)
            def main_ch(t):
                pltpu.sync_copy(table.at[gbuf.at[pl.ds(t * CH, CH)]], rows)
                pltpu.sync_copy(rows, acc.at[sidx.at[t]], add=True)

    def write_out(sum_o):
        @pl.loop(0, 40)
        def wrs(k):
            j = k * 16 + s

            @pl.when(j < HALF // 80)
            def _():
                pltpu.sync_copy(acc.at[pl.ds(j * 80, 80)],
                                sum_o.at[pl.ds(c * HALF + j * 80, 80)])

    def phase(s_arr, g_arr, table, sum_o):
        zero_acc()
        plsc.subcore_barrier()
        scan_pass(s_arr, g_arr, table)
        plsc.subcore_barrier()
        write_out(sum_o)
        plsc.subcore_barrier()

    phase(dst_e, src_e, user_x, sum_i)
    phase(src_e, dst_e, item_x, sum_u)


_sc_call = pl.kernel(
    _sc_body,
    out_type=(
        jax.ShapeDtypeStruct((N_NODE, DE), jnp.float32),
        jax.ShapeDtypeStruct((N_NODE, DE), jnp.float32),
    ),
    mesh=plsc.VectorSubcoreMesh(core_axis_name="c", subcore_axis_name="s"),
    compiler_params=pltpu.CompilerParams(use_tc_tiling_on_sc=False),
    scratch_types=(
        pltpu.VMEM((BLK,), jnp.int32),
        pltpu.VMEM((BLK,), jnp.int32),
        pltpu.VMEM((NCH, CH), jnp.int32),
        pltpu.VMEM((CH, DE), jnp.float32),
        pltpu.VMEM_SHARED((ACC_ROWS, DE), jnp.float32),
    ),
)

BR = 1000


def _dense_body(hu, su, hi, si, wsT, wnT, bb, ou, oi):
    ws = wsT[...]
    wn = wnT[...]
    bv = bb[...]
    sue = su[...]
    agg_u = sue * (1.0 / jnp.maximum(sue[:, 32:33], 1.0))
    ou[...] = (jnp.dot(hu[...], ws, preferred_element_type=jnp.float32)
               + jnp.dot(agg_u, wn, preferred_element_type=jnp.float32) + bv)
    sie = si[...]
    agg_i = sie * (1.0 / jnp.maximum(sie[:, 32:33], 1.0))
    oi[...] = (jnp.dot(hi[...], ws, preferred_element_type=jnp.float32)
               + jnp.dot(agg_i, wn, preferred_element_type=jnp.float32) + bv)


def _spec(cols):
    return pl.BlockSpec((BR, cols), lambda i: (i, 0))


def _full_spec(shape):
    return pl.BlockSpec(shape, lambda i: tuple(0 for _ in shape))


_dense_call = pl.pallas_call(
    _dense_body,
    grid=(N_NODE // BR,),
    in_specs=[
        _spec(D), _spec(DE), _spec(D), _spec(DE),
        _full_spec((D, D)), _full_spec((DE, D)), _full_spec((1, D)),
    ],
    out_specs=[_spec(D), _spec(D)],
    out_shape=(
        jax.ShapeDtypeStruct((N_NODE, D), jnp.float32),
        jax.ShapeDtypeStruct((N_NODE, D), jnp.float32),
    ),
)


def _extend(table):
    one = jnp.ones((N_NODE, 1), jnp.float32)
    pad = jnp.zeros((N_NODE, DE - D - 1), jnp.float32)
    return jnp.concatenate([table, one, pad], axis=1)


def kernel(edge_index, user_table, item_table, W_self, W_neigh, b):
    edge_index = edge_index.astype(jnp.int32)
    zrow = jnp.zeros((3144, DE), jnp.float32)
    sum_i, sum_u = _sc_call(
        edge_index[0], edge_index[1], _extend(user_table),
        _extend(item_table), zrow)
    wnT_ext = jnp.concatenate(
        [W_neigh.T, jnp.zeros((DE - D, D), jnp.float32)], axis=0)
    out_user, out_item = _dense_call(
        user_table, sum_u, item_table, sum_i,
        W_self.T, wnT_ext, b.reshape(1, D))
    return (out_user, out_item)
